# baseline (speedup 1.0000x reference)
import jax
import jax.numpy as jnp
from jax.experimental import pallas as pl
from jax.experimental.pallas import tpu as pltpu

_G = 4


def _loss_kernel(x_ref, m_ref, o_ref, xy_ref):
    c = pl.program_id(0)
    g = pl.program_id(1)

    @pl.when(g == 0)
    def _():
        m = m_ref[0, 0]
        H, W = m.shape
        row = jax.lax.broadcasted_iota(jnp.int32, (H, W), 0)
        col = jax.lax.broadcasted_iota(jnp.int32, (H, W), 1)
        lin = row * W + col
        mv = jnp.max(m)
        idx = jnp.min(jnp.where(m == mv, lin, jnp.int32(H * W)))
        xy_ref[0] = (idx // W).astype(jnp.float32)
        xy_ref[1] = (idx % W).astype(jnp.float32)

    x0 = xy_ref[0]
    y0 = xy_ref[1]

    xb = x_ref[0]
    CB, H, W = xb.shape
    j0 = c * (_G * CB) + g * CB
    jj = (jax.lax.broadcasted_iota(jnp.int32, (CB, H), 0) + j0).astype(jnp.float32)
    kk = jax.lax.broadcasted_iota(jnp.int32, (CB, H), 1).astype(jnp.float32)
    wgt = (x0 - jj) ** 2 + (y0 - kk) ** 2

    r = jnp.sum(jnp.sum(xb * wgt[:, :, None], axis=0), axis=0, keepdims=True)

    @pl.when(g == 0)
    def _():
        o_ref[0] = r

    @pl.when(g > 0)
    def _():
        o_ref[0] += r


def kernel(tensor):
    B, C, H, W = tensor.shape
    CB = C // (2 * _G)
    out = pl.pallas_call(
        _loss_kernel,
        out_shape=jax.ShapeDtypeStruct((2, 1, W), jnp.float32),
        grid=(2, _G),
        in_specs=[
            pl.BlockSpec((1, CB, H, W), lambda c, g: (0, c * _G + g, 0, 0)),
            pl.BlockSpec((1, 1, H, W), lambda c, g: (0, 0, 0, 0)),
        ],
        out_specs=pl.BlockSpec((1, 1, W), lambda c, g: (c, 0, 0)),
        scratch_shapes=[pltpu.SMEM((2,), jnp.float32)],
        compiler_params=pltpu.CompilerParams(
            dimension_semantics=("parallel", "arbitrary"),
        ),
    )(tensor, tensor)
    return out[0, 0] + out[1, 0]


# speedup vs baseline: 1.1571x; 1.1571x over previous
import jax
import jax.numpy as jnp
from jax.experimental import pallas as pl
from jax.experimental.pallas import tpu as pltpu

_G = 8


def _loss_kernel(x_ref, o_ref, xy_ref):
    g = pl.program_id(0)

    xb = x_ref[0]
    CB, H, W = xb.shape

    @pl.when(g == 0)
    def _():
        m = xb[0]
        row = jax.lax.broadcasted_iota(jnp.int32, (H, W), 0)
        col = jax.lax.broadcasted_iota(jnp.int32, (H, W), 1)
        lin = row * W + col
        mv = jnp.max(m)
        idx = jnp.min(jnp.where(m == mv, lin, jnp.int32(H * W)))
        xy_ref[0] = (idx // W).astype(jnp.float32)
        xy_ref[1] = (idx % W).astype(jnp.float32)

    x0 = xy_ref[0]
    y0 = xy_ref[1]

    j0 = g * CB
    jj = (jax.lax.broadcasted_iota(jnp.int32, (CB, H), 0) + j0).astype(jnp.float32)
    kk = jax.lax.broadcasted_iota(jnp.int32, (CB, H), 1).astype(jnp.float32)
    wgt = (x0 - jj) ** 2 + (y0 - kk) ** 2

    r = jnp.sum(jnp.sum(xb * wgt[:, :, None], axis=0), axis=0, keepdims=True)

    @pl.when(g == 0)
    def _():
        o_ref[:] = r

    @pl.when(g > 0)
    def _():
        o_ref[:] += r


def kernel(tensor):
    B, C, H, W = tensor.shape
    CB = C // _G
    out = pl.pallas_call(
        _loss_kernel,
        out_shape=jax.ShapeDtypeStruct((1, W), jnp.float32),
        grid=(_G,),
        in_specs=[pl.BlockSpec((1, CB, H, W), lambda g: (0, g, 0, 0))],
        out_specs=pl.BlockSpec((1, W), lambda g: (0, 0)),
        scratch_shapes=[pltpu.SMEM((2,), jnp.float32)],
        compiler_params=pltpu.CompilerParams(
            dimension_semantics=("arbitrary",),
        ),
    )(tensor)
    return out[0]


# speedup vs baseline: 1.3182x; 1.1392x over previous
import jax
import jax.numpy as jnp
from jax.experimental import pallas as pl
from jax.experimental.pallas import tpu as pltpu

_G = 8


def _loss_kernel(x_ref, o_ref, xy_ref, r_acc, cl_acc):
    g = pl.program_id(0)

    xb = x_ref[0]
    CB, H, W = xb.shape

    @pl.when(g == 0)
    def _():
        m = xb[0]
        row = jax.lax.broadcasted_iota(jnp.int32, (H, W), 0)
        col = jax.lax.broadcasted_iota(jnp.int32, (H, W), 1)
        lin = row * W + col
        mv = jnp.max(m)
        idx = jnp.min(jnp.where(m == mv, lin, jnp.int32(H * W)))
        xy_ref[0] = (idx // W).astype(jnp.float32)
        xy_ref[1] = (idx % W).astype(jnp.float32)

    r_acc[pl.ds(g * CB, CB), :] = jnp.sum(xb, axis=1)

    cl = jnp.sum(xb, axis=0)

    @pl.when(g == 0)
    def _():
        cl_acc[:] = cl

    @pl.when(g > 0)
    def _():
        cl_acc[:] += cl

    @pl.when(g == _G - 1)
    def _():
        x0 = xy_ref[0]
        y0 = xy_ref[1]
        C = _G * CB
        jrow = jax.lax.broadcasted_iota(jnp.int32, (1, C), 1).astype(jnp.float32)
        krow = jax.lax.broadcasted_iota(jnp.int32, (1, H), 1).astype(jnp.float32)
        a = (x0 - jrow) ** 2
        b = (y0 - krow) ** 2
        o_ref[:] = (
            jnp.dot(a, r_acc[:], preferred_element_type=jnp.float32)
            + jnp.dot(b, cl_acc[:], preferred_element_type=jnp.float32)
        )


def kernel(tensor):
    B, C, H, W = tensor.shape
    CB = C // _G
    out = pl.pallas_call(
        _loss_kernel,
        out_shape=jax.ShapeDtypeStruct((1, W), jnp.float32),
        grid=(_G,),
        in_specs=[pl.BlockSpec((1, CB, H, W), lambda g: (0, g, 0, 0))],
        out_specs=pl.BlockSpec((1, W), lambda g: (0, 0)),
        scratch_shapes=[
            pltpu.SMEM((2,), jnp.float32),
            pltpu.VMEM((C, W), jnp.float32),
            pltpu.VMEM((H, W), jnp.float32),
        ],
        compiler_params=pltpu.CompilerParams(
            dimension_semantics=("arbitrary",),
        ),
    )(tensor)
    return out[0]
